# baseline (speedup 1.0000x reference)
import jax
import jax.numpy as jnp
from jax.experimental import pallas as pl
from jax.experimental.pallas import tpu as pltpu

K = 1024
C = 64
HW = 1024


def _vq_image_block(x_ref, cb_ref, qout_ref, idx_ref, loss_ref):
    cb = cb_ref[...]
    e_sq = jnp.sum(cb * cb, axis=-1)
    cbm2 = cb * -2.0
    loss_acc = jnp.zeros((), jnp.float32)
    for u in range(2):
        xb = x_ref[u]
        xpx = jnp.transpose(xb)
        s = jax.lax.dot_general(
            xpx, cbm2, (((1,), (1,)), ((), ())),
            preferred_element_type=jnp.float32)
        s = s + e_sq[None, :]
        m = jnp.min(s, axis=-1)
        idx = jnp.argmin(s, axis=-1)
        onehot = (jax.lax.broadcasted_iota(jnp.int32, (HW, K), 1)
                  == idx[:, None]).astype(jnp.float32)
        quant_t = jax.lax.dot_general(
            cb, onehot, (((0,), (1,)), ((), ())),
            preferred_element_type=jnp.float32)
        qout_ref[u] = quant_t.reshape(C, 32, 32)
        idx_ref[0, u, :] = idx
        loss_acc = loss_acc + (jnp.sum(m) + jnp.sum(xb * xb))
    loss_ref[0, 0, :] = jnp.broadcast_to(loss_acc, (128,))


@jax.jit
def kernel(x, codebook):
    B, Cc, H, W = x.shape
    x3 = x.reshape(B, Cc, H * W)
    qout, idx3, loss3 = pl.pallas_call(
        _vq_image_block,
        grid=(B // 2,),
        in_specs=[
            pl.BlockSpec((2, Cc, H * W), lambda i: (i, 0, 0)),
            pl.BlockSpec((K, C), lambda i: (0, 0)),
        ],
        out_specs=[
            pl.BlockSpec((2, Cc, H, W), lambda i: (i, 0, 0, 0)),
            pl.BlockSpec((1, 2, H * W), lambda i: (i, 0, 0)),
            pl.BlockSpec((1, 1, 128), lambda i: (i, 0, 0)),
        ],
        out_shape=[
            jax.ShapeDtypeStruct((B, Cc, H, W), jnp.float32),
            jax.ShapeDtypeStruct((B // 2, 2, H * W), jnp.int32),
            jax.ShapeDtypeStruct((B // 2, 1, 128), jnp.float32),
        ],
        compiler_params=pltpu.CompilerParams(
            dimension_semantics=("parallel",)),
    )(x3, codebook)
    loss = (jnp.sum(loss3[:, 0, 0]) / (B * Cc * H * W)).astype(jnp.float32)
    idx_emb = idx3.reshape(B, H * W)
    return (qout, loss, loss, idx_emb)


# speedup vs baseline: 1.1823x; 1.1823x over previous
import jax
import jax.numpy as jnp
from jax.experimental import pallas as pl
from jax.experimental.pallas import tpu as pltpu

K = 1024
C = 64
HW = 1024


def _vq_image_block(x_ref, cb_ref, qout_ref, idx_ref, loss_ref):
    cb = cb_ref[...]
    e_sq = jnp.sum(cb * cb, axis=-1)
    cbm2 = cb * -2.0
    loss_acc = jnp.zeros((), jnp.float32)
    for u in range(2):
        xb = x_ref[u]
        xpx = jnp.transpose(xb)
        s = jax.lax.dot_general(
            xpx, cbm2, (((1,), (1,)), ((), ())),
            preferred_element_type=jnp.float32)
        s = s + e_sq[None, :]
        m = jnp.min(s, axis=-1)
        idx = jnp.argmin(s, axis=-1)
        onehot = (jax.lax.broadcasted_iota(jnp.int32, (HW, K), 1)
                  == idx[:, None]).astype(jnp.float32)
        quant_t = jax.lax.dot_general(
            cb, onehot, (((0,), (1,)), ((), ())),
            preferred_element_type=jnp.float32)
        qout_ref[u] = quant_t
        idx_ref[0, u, :] = idx
        loss_acc = loss_acc + (jnp.sum(m) + jnp.sum(xb * xb))
    loss_ref[0, 0, :] = jnp.broadcast_to(loss_acc, (128,))


@jax.jit
def kernel(x, codebook):
    B, Cc, H, W = x.shape
    x3 = x.reshape(B, Cc, H * W)
    qout, idx3, loss3 = pl.pallas_call(
        _vq_image_block,
        grid=(B // 2,),
        in_specs=[
            pl.BlockSpec((2, Cc, H * W), lambda i: (i, 0, 0)),
            pl.BlockSpec((K, C), lambda i: (0, 0)),
        ],
        out_specs=[
            pl.BlockSpec((2, Cc, H * W), lambda i: (i, 0, 0)),
            pl.BlockSpec((1, 2, H * W), lambda i: (i, 0, 0)),
            pl.BlockSpec((1, 1, 128), lambda i: (i, 0, 0)),
        ],
        out_shape=[
            jax.ShapeDtypeStruct((B, Cc, H * W), jnp.float32),
            jax.ShapeDtypeStruct((B // 2, 2, H * W), jnp.int32),
            jax.ShapeDtypeStruct((B // 2, 1, 128), jnp.float32),
        ],
        compiler_params=pltpu.CompilerParams(
            dimension_semantics=("parallel",)),
    )(x3, codebook)
    loss = (jnp.sum(loss3[:, 0, 0]) / (B * Cc * H * W)).astype(jnp.float32)
    quant_out = qout.reshape(B, Cc, H, W)
    idx_emb = idx3.reshape(B, H * W)
    return (quant_out, loss, loss, idx_emb)


# speedup vs baseline: 1.2794x; 1.0821x over previous
import jax
import jax.numpy as jnp
from jax.experimental import pallas as pl
from jax.experimental.pallas import tpu as pltpu

K = 1024
C = 64
HW = 1024


def _vq_image_block(x_ref, cb_ref, qout_ref, idx_ref, loss_ref):
    cb = cb_ref[...]
    e_sq = jnp.sum(cb * cb, axis=-1)
    cbm2 = cb * -2.0
    loss_acc = jnp.zeros((), jnp.float32)
    for u in range(2):
        xb = x_ref[u]
        xpx = jnp.transpose(xb)
        s = jax.lax.dot_general(
            xpx, cbm2, (((1,), (1,)), ((), ())),
            preferred_element_type=jnp.float32)
        s = s + e_sq[None, :]
        idx = jnp.argmin(s, axis=-1)
        onehot = (jax.lax.broadcasted_iota(jnp.int32, (HW, K), 1)
                  == idx[:, None]).astype(jnp.float32)
        quant_t = jax.lax.dot_general(
            cb, onehot, (((0,), (1,)), ((), ())),
            preferred_element_type=jnp.float32)
        qout_ref[u] = quant_t
        idx_ref[0, u, :] = idx
        loss_acc = loss_acc + jnp.sum((xb - quant_t) ** 2)
    loss_ref[0, 0, :] = jnp.broadcast_to(loss_acc, (128,))


@jax.jit
def kernel(x, codebook):
    B, Cc, H, W = x.shape
    x3 = x.reshape(B, Cc, H * W)
    qout, idx3, loss3 = pl.pallas_call(
        _vq_image_block,
        grid=(B // 2,),
        in_specs=[
            pl.BlockSpec((2, Cc, H * W), lambda i: (i, 0, 0)),
            pl.BlockSpec((K, C), lambda i: (0, 0)),
        ],
        out_specs=[
            pl.BlockSpec((2, Cc, H * W), lambda i: (i, 0, 0)),
            pl.BlockSpec((1, 2, H * W), lambda i: (i, 0, 0)),
            pl.BlockSpec((1, 1, 128), lambda i: (i, 0, 0)),
        ],
        out_shape=[
            jax.ShapeDtypeStruct((B, Cc, H * W), jnp.float32),
            jax.ShapeDtypeStruct((B // 2, 2, H * W), jnp.int32),
            jax.ShapeDtypeStruct((B // 2, 1, 128), jnp.float32),
        ],
        compiler_params=pltpu.CompilerParams(
            dimension_semantics=("parallel",)),
    )(x3, codebook)
    loss = (jnp.sum(loss3[:, 0, 0]) / (B * Cc * H * W)).astype(jnp.float32)
    quant_out = qout.reshape(B, Cc, H, W)
    idx_emb = idx3.reshape(B, H * W)
    return (quant_out, loss, loss, idx_emb)


# speedup vs baseline: 1.3540x; 1.0584x over previous
import jax
import jax.numpy as jnp
from jax.experimental import pallas as pl
from jax.experimental.pallas import tpu as pltpu

K = 1024
C = 64
HW = 1024


def _vq_image_block(x_ref, cb_ref, qout_ref, idx_ref, loss_ref):
    cb = cb_ref[...]
    e_sq = jnp.sum(cb * cb, axis=-1)
    cbm2 = cb * -2.0
    loss_acc = jnp.zeros((), jnp.float32)
    for u in range(4):
        xb = x_ref[u]
        xpx = jnp.transpose(xb)
        s = jax.lax.dot_general(
            xpx, cbm2, (((1,), (1,)), ((), ())),
            preferred_element_type=jnp.float32)
        s = s + e_sq[None, :]
        idx = jnp.argmin(s, axis=-1)
        onehot = (jax.lax.broadcasted_iota(jnp.int32, (HW, K), 1)
                  == idx[:, None]).astype(jnp.float32)
        quant_t = jax.lax.dot_general(
            cb, onehot, (((0,), (1,)), ((), ())),
            preferred_element_type=jnp.float32)
        qout_ref[u] = quant_t
        idx_ref[0, u, :] = idx
        loss_acc = loss_acc + jnp.sum((xb - quant_t) ** 2)
    loss_ref[0, 0, :] = jnp.broadcast_to(loss_acc, (128,))


@jax.jit
def kernel(x, codebook):
    B, Cc, H, W = x.shape
    x3 = x.reshape(B, Cc, H * W)
    qout, idx3, loss3 = pl.pallas_call(
        _vq_image_block,
        grid=(B // 4,),
        in_specs=[
            pl.BlockSpec((4, Cc, H * W), lambda i: (i, 0, 0)),
            pl.BlockSpec((K, C), lambda i: (0, 0)),
        ],
        out_specs=[
            pl.BlockSpec((4, Cc, H * W), lambda i: (i, 0, 0)),
            pl.BlockSpec((1, 4, H * W), lambda i: (i, 0, 0)),
            pl.BlockSpec((1, 1, 128), lambda i: (i, 0, 0)),
        ],
        out_shape=[
            jax.ShapeDtypeStruct((B, Cc, H * W), jnp.float32),
            jax.ShapeDtypeStruct((B // 4, 4, H * W), jnp.int32),
            jax.ShapeDtypeStruct((B // 4, 1, 128), jnp.float32),
        ],
        compiler_params=pltpu.CompilerParams(
            dimension_semantics=("parallel",)),
    )(x3, codebook)
    loss = (jnp.sum(loss3[:, 0, 0]) / (B * Cc * H * W)).astype(jnp.float32)
    quant_out = qout.reshape(B, Cc, H, W)
    idx_emb = idx3.reshape(B, H * W)
    return (quant_out, loss, loss, idx_emb)


# speedup vs baseline: 1.5769x; 1.1646x over previous
import jax
import jax.numpy as jnp
from jax.experimental import pallas as pl
from jax.experimental.pallas import tpu as pltpu

K = 1024
C = 64
HW = 1024


def _vq_image_block(x_ref, cb_ref, qout_ref, idx_ref, loss_ref):
    cb = cb_ref[...]
    e_sq = jnp.sum(cb * cb, axis=-1)
    cbm2 = cb * -2.0
    loss_acc = jnp.zeros((), jnp.float32)
    for u in range(8):
        xb = x_ref[u]
        xpx = jnp.transpose(xb)
        s = jax.lax.dot_general(
            xpx, cbm2, (((1,), (1,)), ((), ())),
            preferred_element_type=jnp.float32)
        s = s + e_sq[None, :]
        idx = jnp.argmin(s, axis=-1)
        onehot = (jax.lax.broadcasted_iota(jnp.int32, (HW, K), 1)
                  == idx[:, None]).astype(jnp.float32)
        quant_t = jax.lax.dot_general(
            cb, onehot, (((0,), (1,)), ((), ())),
            preferred_element_type=jnp.float32)
        qout_ref[u] = quant_t
        idx_ref[0, u, :] = idx
        loss_acc = loss_acc + jnp.sum((xb - quant_t) ** 2)
    loss_ref[0, 0, :] = jnp.broadcast_to(loss_acc, (128,))


@jax.jit
def kernel(x, codebook):
    B, Cc, H, W = x.shape
    x3 = x.reshape(B, Cc, H * W)
    qout, idx3, loss3 = pl.pallas_call(
        _vq_image_block,
        grid=(B // 8,),
        in_specs=[
            pl.BlockSpec((8, Cc, H * W), lambda i: (i, 0, 0)),
            pl.BlockSpec((K, C), lambda i: (0, 0)),
        ],
        out_specs=[
            pl.BlockSpec((8, Cc, H * W), lambda i: (i, 0, 0)),
            pl.BlockSpec((1, 8, H * W), lambda i: (i, 0, 0)),
            pl.BlockSpec((1, 1, 128), lambda i: (i, 0, 0)),
        ],
        out_shape=[
            jax.ShapeDtypeStruct((B, Cc, H * W), jnp.float32),
            jax.ShapeDtypeStruct((B // 8, 8, H * W), jnp.int32),
            jax.ShapeDtypeStruct((B // 8, 1, 128), jnp.float32),
        ],
        compiler_params=pltpu.CompilerParams(
            dimension_semantics=("parallel",)),
    )(x3, codebook)
    loss = (jnp.sum(loss3[:, 0, 0]) / (B * Cc * H * W)).astype(jnp.float32)
    quant_out = qout.reshape(B, Cc, H, W)
    idx_emb = idx3.reshape(B, H * W)
    return (quant_out, loss, loss, idx_emb)
